# baseline (speedup 1.0000x reference)
import functools

import jax
import jax.numpy as jnp
from jax import lax
from jax.experimental import pallas as pl
from jax.experimental.pallas import tpu as pltpu
from jax.experimental.pallas import tpu_sc as plsc

_S = 10.0
_NC = 2
_NS = 16
_NW = _NC * _NS
_LANES = 16


def _sc_gather_pos(costh_flat, label, B, C):
  bpw = B // _NW
  mesh = plsc.VectorSubcoreMesh(core_axis_name="c", subcore_axis_name="s")

  @functools.partial(
      pl.kernel,
      mesh=mesh,
      out_type=jax.ShapeDtypeStruct((B,), jnp.float32),
      scratch_types=[
          pltpu.VMEM((bpw,), jnp.int32),
          pltpu.VMEM((bpw,), jnp.int32),
          pltpu.VMEM((bpw,), jnp.float32),
          pltpu.SemaphoreType.DMA,
      ],
  )
  def gather_kernel(costh_hbm, label_hbm, out_hbm, lbl_v, idx_v, pos_v, sem):
    wid = lax.axis_index("s") * _NC + lax.axis_index("c")
    base = wid * bpw
    pltpu.sync_copy(label_hbm.at[pl.ds(base, bpw)], lbl_v)

    def body(j, carry):
      lbl16 = lbl_v[pl.ds(j * _LANES, _LANES)]
      row16 = base + j * _LANES + lax.iota(jnp.int32, _LANES)
      idx_v[pl.ds(j * _LANES, _LANES)] = lbl16 + row16 * C
      return carry

    lax.fori_loop(0, bpw // _LANES, body, 0)
    pltpu.async_copy(costh_hbm.at[idx_v], pos_v, sem).wait()
    pltpu.sync_copy(pos_v, out_hbm.at[pl.ds(base, bpw)])

  return gather_kernel(costh_flat, label)


def _tc_loss_sum(costh, pos_col, B, C, R):
  G = B // R
  scale = 1.0 / (float(B) * float(C - 1))

  def body(x_ref, p_ref, o_ref):
    i = pl.program_id(0)

    @pl.when(i == 0)
    def _():
      o_ref[...] = jnp.zeros((1, 1), jnp.float32)

    x = x_ref[...]
    p = p_ref[...]
    s = jax.nn.sigmoid(_S * (p - x))
    part = jnp.sum(s) - 0.5 * R
    o_ref[...] += (part * scale).reshape(1, 1)

  return pl.pallas_call(
      body,
      grid=(G,),
      in_specs=[
          pl.BlockSpec((R, C), lambda i: (i, 0)),
          pl.BlockSpec((R, 1), lambda i: (i, 0)),
      ],
      out_specs=pl.BlockSpec((1, 1), lambda i: (0, 0)),
      out_shape=jax.ShapeDtypeStruct((1, 1), jnp.float32),
  )(costh, pos_col)


def _tc_loss_fused(costh, label_col, B, C, R):
  G = B // R
  scale = 1.0 / (float(B) * float(C - 1))

  def body(x_ref, l_ref, o_ref):
    i = pl.program_id(0)

    @pl.when(i == 0)
    def _():
      o_ref[...] = jnp.zeros((1, 1), jnp.float32)

    x = x_ref[...]
    lbl = l_ref[...]
    col = jax.lax.broadcasted_iota(jnp.int32, (R, C), 1)
    p = jnp.sum(jnp.where(col == lbl, x, 0.0), axis=1, keepdims=True)
    s = jax.nn.sigmoid(_S * (p - x))
    part = jnp.sum(s) - 0.5 * R
    o_ref[...] += (part * scale).reshape(1, 1)

  return pl.pallas_call(
      body,
      grid=(G,),
      in_specs=[
          pl.BlockSpec((R, C), lambda i: (i, 0)),
          pl.BlockSpec((R, 1), lambda i: (i, 0)),
      ],
      out_specs=pl.BlockSpec((1, 1), lambda i: (0, 0)),
      out_shape=jax.ShapeDtypeStruct((1, 1), jnp.float32),
  )(costh, label_col)


def kernel(costh, label):
  B, C = costh.shape
  label = label.astype(jnp.int32)
  out = _tc_loss_fused(costh, label.reshape(B, 1), B, C, R=512)
  return out[0, 0]


# speedup vs baseline: 1.0831x; 1.0831x over previous
import functools

import jax
import jax.numpy as jnp
from jax import lax
from jax.experimental import pallas as pl
from jax.experimental.pallas import tpu as pltpu
from jax.experimental.pallas import tpu_sc as plsc

_S = 10.0
_NC = 2
_NS = 16
_NW = _NC * _NS
_LANES = 16


def _sc_gather_pos(costh_flat, label, B, C):
  bpw = B // _NW
  mesh = plsc.VectorSubcoreMesh(core_axis_name="c", subcore_axis_name="s")

  @functools.partial(
      pl.kernel,
      mesh=mesh,
      out_type=jax.ShapeDtypeStruct((B,), jnp.float32),
      scratch_types=[
          pltpu.VMEM((bpw,), jnp.int32),
          pltpu.VMEM((bpw,), jnp.int32),
          pltpu.VMEM((bpw,), jnp.float32),
          pltpu.SemaphoreType.DMA,
      ],
  )
  def gather_kernel(costh_hbm, label_hbm, out_hbm, lbl_v, idx_v, pos_v, sem):
    wid = lax.axis_index("s") * _NC + lax.axis_index("c")
    base = wid * bpw
    pltpu.sync_copy(label_hbm.at[pl.ds(base, bpw)], lbl_v)

    def body(j, carry):
      lbl16 = lbl_v[pl.ds(j * _LANES, _LANES)]
      row16 = base + j * _LANES + lax.iota(jnp.int32, _LANES)
      idx_v[pl.ds(j * _LANES, _LANES)] = lbl16 + row16 * C
      return carry

    lax.fori_loop(0, bpw // _LANES, body, 0)
    pltpu.async_copy(costh_hbm.at[idx_v], pos_v, sem).wait()
    pltpu.sync_copy(pos_v, out_hbm.at[pl.ds(base, bpw)])

  return gather_kernel(costh_flat, label)


def _tc_loss_sum(costh, pos_col, B, C, R):
  G = B // R
  scale = 1.0 / (float(B) * float(C - 1))

  def body(x_ref, p_ref, o_ref):
    i = pl.program_id(0)

    @pl.when(i == 0)
    def _():
      o_ref[...] = jnp.zeros((1, 1), jnp.float32)

    x = x_ref[...]
    p = p_ref[...]
    s = jax.nn.sigmoid(_S * (p - x))
    part = jnp.sum(s) - 0.5 * R
    o_ref[...] += (part * scale).reshape(1, 1)

  return pl.pallas_call(
      body,
      grid=(G,),
      in_specs=[
          pl.BlockSpec((R, C), lambda i: (i, 0)),
          pl.BlockSpec((R, 1), lambda i: (i, 0)),
      ],
      out_specs=pl.BlockSpec((1, 1), lambda i: (0, 0)),
      out_shape=jax.ShapeDtypeStruct((1, 1), jnp.float32),
  )(costh, pos_col)


def _tc_loss_fused(costh, label_col, B, C, R):
  G = B // R
  scale = 1.0 / (float(B) * float(C - 1))

  def body(x_ref, l_ref, o_ref):
    i = pl.program_id(0)

    @pl.when(i == 0)
    def _():
      o_ref[...] = jnp.zeros((1, 1), jnp.float32)

    x = x_ref[...]
    lbl = l_ref[...]
    col = jax.lax.broadcasted_iota(jnp.int32, (R, C), 1)
    p = jnp.sum(jnp.where(col == lbl, x, 0.0), axis=1, keepdims=True)
    s = jax.nn.sigmoid(_S * (p - x))
    part = jnp.sum(s) - 0.5 * R
    o_ref[...] += (part * scale).reshape(1, 1)

  return pl.pallas_call(
      body,
      grid=(G,),
      in_specs=[
          pl.BlockSpec((R, C), lambda i: (i, 0)),
          pl.BlockSpec((R, 1), lambda i: (i, 0)),
      ],
      out_specs=pl.BlockSpec((1, 1), lambda i: (0, 0)),
      out_shape=jax.ShapeDtypeStruct((1, 1), jnp.float32),
  )(costh, label_col)


def kernel(costh, label):
  B, C = costh.shape
  label = label.astype(jnp.int32)
  out = _tc_loss_fused(costh, label.reshape(B, 1), B, C, R=4096)
  return out[0, 0]


# speedup vs baseline: 1.1789x; 1.0885x over previous
import functools

import jax
import jax.numpy as jnp
from jax import lax
from jax.experimental import pallas as pl
from jax.experimental.pallas import tpu as pltpu
from jax.experimental.pallas import tpu_sc as plsc

_S = 10.0
_NC = 2
_NS = 16
_NW = _NC * _NS
_LANES = 16


def _sc_gather_pos(costh_flat, label, B, C):
  bpw = B // _NW
  mesh = plsc.VectorSubcoreMesh(core_axis_name="c", subcore_axis_name="s")

  @functools.partial(
      pl.kernel,
      mesh=mesh,
      out_type=jax.ShapeDtypeStruct((B,), jnp.float32),
      scratch_types=[
          pltpu.VMEM((bpw,), jnp.int32),
          pltpu.VMEM((bpw,), jnp.int32),
          pltpu.VMEM((bpw,), jnp.float32),
          pltpu.SemaphoreType.DMA,
      ],
  )
  def gather_kernel(costh_hbm, label_hbm, out_hbm, lbl_v, idx_v, pos_v, sem):
    wid = lax.axis_index("s") * _NC + lax.axis_index("c")
    base = wid * bpw
    pltpu.sync_copy(label_hbm.at[pl.ds(base, bpw)], lbl_v)

    def body(j, carry):
      lbl16 = lbl_v[pl.ds(j * _LANES, _LANES)]
      row16 = base + j * _LANES + lax.iota(jnp.int32, _LANES)
      idx_v[pl.ds(j * _LANES, _LANES)] = lbl16 + row16 * C
      return carry

    lax.fori_loop(0, bpw // _LANES, body, 0)
    pltpu.async_copy(costh_hbm.at[idx_v], pos_v, sem).wait()
    pltpu.sync_copy(pos_v, out_hbm.at[pl.ds(base, bpw)])

  return gather_kernel(costh_flat, label)


def _tc_loss_sum(costh, pos_col, B, C, R):
  G = B // R
  scale = 1.0 / (float(B) * float(C - 1))

  def body(x_ref, p_ref, o_ref):
    i = pl.program_id(0)

    @pl.when(i == 0)
    def _():
      o_ref[...] = jnp.zeros((1, 1), jnp.float32)

    x = x_ref[...]
    p = p_ref[...]
    s = jax.nn.sigmoid(_S * (p - x))
    part = jnp.sum(s) - 0.5 * R
    o_ref[...] += (part * scale).reshape(1, 1)

  return pl.pallas_call(
      body,
      grid=(G,),
      in_specs=[
          pl.BlockSpec((R, C), lambda i: (i, 0)),
          pl.BlockSpec((R, 1), lambda i: (i, 0)),
      ],
      out_specs=pl.BlockSpec((1, 1), lambda i: (0, 0)),
      out_shape=jax.ShapeDtypeStruct((1, 1), jnp.float32),
  )(costh, pos_col)


def _tc_loss_fused(costh, label_col, B, C, R):
  G = B // R
  scale = 1.0 / (float(B) * float(C - 1))

  def body(x_ref, l_ref, o_ref):
    i = pl.program_id(0)

    @pl.when(i == 0)
    def _():
      o_ref[...] = jnp.zeros((1, 1), jnp.float32)

    x = x_ref[...]
    lbl = l_ref[...]
    col = jax.lax.broadcasted_iota(jnp.int32, (R, C), 1)
    p = jnp.sum(jnp.where(col == lbl, x, 0.0), axis=1, keepdims=True)
    s = jax.nn.sigmoid(_S * (p - x))
    part = jnp.sum(s) - 0.5 * R
    o_ref[...] += (part * scale).reshape(1, 1)

  return pl.pallas_call(
      body,
      grid=(G,),
      in_specs=[
          pl.BlockSpec((R, C), lambda i: (i, 0)),
          pl.BlockSpec((R, 1), lambda i: (i, 0)),
      ],
      out_specs=pl.BlockSpec((1, 1), lambda i: (0, 0)),
      out_shape=jax.ShapeDtypeStruct((1, 1), jnp.float32),
  )(costh, label_col)


def _tc_loss_fused_multi(costh, label_col, B, C, R, K):
  G = B // (R * K)
  scale = 1.0 / (float(B) * float(C - 1))

  def body(*refs):
    o_ref = refs[-1]
    i = pl.program_id(0)

    @pl.when(i == 0)
    def _():
      o_ref[...] = jnp.zeros((1, 1), jnp.float32)

    col = jax.lax.broadcasted_iota(jnp.int32, (R, C), 1)
    acc = jnp.zeros((), jnp.float32)
    for m in range(K):
      x = refs[2 * m][...]
      lbl = refs[2 * m + 1][...]
      p = jnp.sum(jnp.where(col == lbl, x, 0.0), axis=1, keepdims=True)
      s = jax.nn.sigmoid(_S * (p - x))
      acc += jnp.sum(s) - 0.5 * R
    o_ref[...] += (acc * scale).reshape(1, 1)

  def xmap(m):
    return lambda i: (i + m * G, 0)

  in_specs = []
  operands = []
  for m in range(K):
    in_specs.append(pl.BlockSpec((R, C), xmap(m)))
    in_specs.append(pl.BlockSpec((R, 1), xmap(m)))
    operands.extend([costh, label_col])

  return pl.pallas_call(
      body,
      grid=(G,),
      in_specs=in_specs,
      out_specs=pl.BlockSpec((1, 1), lambda i: (0, 0)),
      out_shape=jax.ShapeDtypeStruct((1, 1), jnp.float32),
  )(*operands)


def kernel(costh, label):
  B, C = costh.shape
  label = label.astype(jnp.int32)
  out = _tc_loss_fused_multi(costh, label.reshape(B, 1), B, C, R=2048, K=2)
  return out[0, 0]
